# baseline (speedup 1.0000x reference)
import functools

import jax
import jax.numpy as jnp
from jax import lax
from jax.experimental import pallas as pl
from jax.experimental.pallas import tpu as pltpu
from jax.experimental.pallas import tpu_sc as plsc

N_EXPERTS = 8
TOP_K = 2
ALPHA = 0.001
TOKEN_BLOCK = 1024
NUM_CORES = 2
NUM_SUBCORES = 16
NUM_WORKERS = NUM_CORES * NUM_SUBCORES
LANES = 16


def _gate_block(x_ref, w_ref, p_ref, aux_ref, acc_ref):
    i = pl.program_id(0)
    nb = pl.num_programs(0)

    x = x_ref[...]
    w = w_ref[...]
    logits = lax.dot_general(
        w, x, (((1,), (1,)), ((), ())), preferred_element_type=jnp.float32
    )

    m = jnp.max(logits, axis=0, keepdims=True)
    e = jnp.exp(logits - m)
    p = e / jnp.sum(e, axis=0, keepdims=True)
    p_ref[0, :, :] = p

    iota = lax.broadcasted_iota(jnp.int32, p.shape, 0)
    m1 = jnp.max(p, axis=0, keepdims=True)
    idx1 = jnp.min(jnp.where(p == m1, iota, N_EXPERTS), axis=0, keepdims=True)
    is1 = iota == idx1
    p2 = jnp.where(is1, -1.0, p)
    m2 = jnp.max(p2, axis=0, keepdims=True)
    idx2 = jnp.min(jnp.where(p2 == m2, iota, N_EXPERTS), axis=0, keepdims=True)
    is2 = iota == idx2

    part = jnp.concatenate(
        [p, jnp.where(is1 | is2, 1.0, 0.0)], axis=0
    )

    @pl.when(i == 0)
    def _init():
        acc_ref[...] = part

    @pl.when(i > 0)
    def _acc():
        acc_ref[...] += part

    @pl.when(i == nb - 1)
    def _fin():
        acc = jnp.sum(acc_ref[...], axis=1)
        total = nb * x.shape[0]
        pi = acc[:N_EXPERTS] / total
        fi = acc[N_EXPERTS:] * (N_EXPERTS / (total * TOP_K))
        aux = jnp.sum(pi * fi) * ALPHA
        aux_ref[...] = jnp.full((8, 128), aux, jnp.float32)


def _scores_kernel(hs, w):
    t, h = hs.shape
    tb = TOKEN_BLOCK
    nb = t // tb
    return pl.pallas_call(
        _gate_block,
        grid=(nb,),
        in_specs=[
            pl.BlockSpec((tb, h), lambda i: (i, 0)),
            pl.BlockSpec((N_EXPERTS, h), lambda i: (0, 0)),
        ],
        out_specs=[
            pl.BlockSpec((1, N_EXPERTS, tb), lambda i: (i, 0, 0)),
            pl.BlockSpec((8, 128), lambda i: (0, 0)),
        ],
        out_shape=[
            jax.ShapeDtypeStruct((nb, N_EXPERTS, tb), jnp.float32),
            jax.ShapeDtypeStruct((8, 128), jnp.float32),
        ],
        scratch_shapes=[pltpu.VMEM((2 * N_EXPERTS, tb), jnp.float32)],
        compiler_params=pltpu.CompilerParams(
            dimension_semantics=("arbitrary",),
        ),
    )(hs, w)


def _route_body(p_hbm, out_hbm, p_v, out_v):
    wid = lax.axis_index("s") * NUM_CORES + lax.axis_index("c")
    pltpu.sync_copy(p_hbm.at[wid], p_v)

    @plsc.parallel_loop(0, TOKEN_BLOCK // LANES, unroll=8)
    def chunk(j):
        sl = pl.ds(j * LANES, LANES)
        pe = [p_v[e, sl] for e in range(N_EXPERTS)]

        best = pe[0]
        bi = jnp.zeros((LANES,), jnp.int32)
        b2 = pe[1]
        bi2 = jnp.full((LANES,), 1, jnp.int32)
        upd = pe[1] > best
        best, b2 = jnp.where(upd, pe[1], best), jnp.where(upd, best, pe[1])
        bi, bi2 = jnp.where(upd, 1, bi), jnp.where(upd, bi, 1)
        for e in range(2, N_EXPERTS):
            upd1 = pe[e] > best
            upd2 = pe[e] > b2
            b2 = jnp.where(upd1, best, jnp.where(upd2, pe[e], b2))
            bi2 = jnp.where(upd1, bi, jnp.where(upd2, e, bi2))
            best = jnp.where(upd1, pe[e], best)
            bi = jnp.where(upd1, e, bi)

        inv = 1.0 / (best + b2 + 1e-20)
        out_v[0, sl] = bi.astype(jnp.float32)
        out_v[1, sl] = bi2.astype(jnp.float32)
        out_v[2, sl] = best * inv
        out_v[3, sl] = b2 * inv

    pltpu.sync_copy(out_v, out_hbm.at[wid])


def _route_sc(scores):
    nw = scores.shape[0]
    tpw = scores.shape[2]
    mesh = plsc.VectorSubcoreMesh(core_axis_name="c", subcore_axis_name="s")
    run = functools.partial(
        pl.kernel,
        mesh=mesh,
        out_type=jax.ShapeDtypeStruct((nw, 2 * TOP_K, tpw), jnp.float32),
        scratch_types=[
            pltpu.VMEM((N_EXPERTS, tpw), jnp.float32),
            pltpu.VMEM((2 * TOP_K, tpw), jnp.float32),
        ],
    )(_route_body)
    return run(scores)


def kernel(hidden_states, kernel):
    bsz, seq_len, h = hidden_states.shape
    t = bsz * seq_len
    hs = hidden_states.reshape(t, h)

    scores, aux = _scores_kernel(hs, kernel)
    route = _route_sc(scores)

    rt = route.transpose(0, 2, 1)
    topk_idx = rt[:, :, :TOP_K].astype(jnp.int32).reshape(t, TOP_K)
    topk_weight = rt[:, :, TOP_K:].reshape(t, TOP_K)
    aux_loss = aux[0, 0]
    return (topk_idx, topk_weight, aux_loss)


# speedup vs baseline: 1.0326x; 1.0326x over previous
import functools

import jax
import jax.numpy as jnp
from jax import lax
from jax.experimental import pallas as pl
from jax.experimental.pallas import tpu as pltpu
from jax.experimental.pallas import tpu_sc as plsc

N_EXPERTS = 8
TOP_K = 2
ALPHA = 0.001
TOKEN_BLOCK = 1024
NUM_CORES = 2
NUM_SUBCORES = 16
NUM_WORKERS = NUM_CORES * NUM_SUBCORES
LANES = 16


def _gate_block(x_ref, w_ref, p_ref, aux_ref, acc_ref):
    i = pl.program_id(0)
    nb = pl.num_programs(0)

    x = x_ref[...]
    w = w_ref[...]
    logits = lax.dot_general(
        w, x, (((1,), (1,)), ((), ())), preferred_element_type=jnp.float32
    )

    m = jnp.max(logits, axis=0, keepdims=True)
    e = jnp.exp(logits - m)
    p = e / jnp.sum(e, axis=0, keepdims=True)
    p_ref[0, :, :] = p

    iota = lax.broadcasted_iota(jnp.int32, p.shape, 0)
    m1 = jnp.max(p, axis=0, keepdims=True)
    idx1 = jnp.min(jnp.where(p == m1, iota, N_EXPERTS), axis=0, keepdims=True)
    is1 = iota == idx1
    p2 = jnp.where(is1, -1.0, p)
    m2 = jnp.max(p2, axis=0, keepdims=True)
    idx2 = jnp.min(jnp.where(p2 == m2, iota, N_EXPERTS), axis=0, keepdims=True)
    is2 = iota == idx2

    part = jnp.concatenate(
        [p, jnp.where(is1 | is2, 1.0, 0.0)], axis=0
    )

    @pl.when(i == 0)
    def _init():
        acc_ref[...] = part

    @pl.when(i > 0)
    def _acc():
        acc_ref[...] += part

    @pl.when(i == nb - 1)
    def _fin():
        acc = jnp.sum(acc_ref[...], axis=1)
        total = nb * x.shape[0]
        pi = acc[:N_EXPERTS] / total
        fi = acc[N_EXPERTS:] * (N_EXPERTS / (total * TOP_K))
        aux = jnp.sum(pi * fi) * ALPHA
        aux_ref[...] = jnp.full((8, 128), aux, jnp.float32)


def _scores_kernel(hs, w):
    t, h = hs.shape
    tb = TOKEN_BLOCK
    nb = t // tb
    return pl.pallas_call(
        _gate_block,
        grid=(nb,),
        in_specs=[
            pl.BlockSpec((tb, h), lambda i: (i, 0)),
            pl.BlockSpec((N_EXPERTS, h), lambda i: (0, 0)),
        ],
        out_specs=[
            pl.BlockSpec((1, N_EXPERTS, tb), lambda i: (i, 0, 0)),
            pl.BlockSpec((8, 128), lambda i: (0, 0)),
        ],
        out_shape=[
            jax.ShapeDtypeStruct((nb, N_EXPERTS, tb), jnp.float32),
            jax.ShapeDtypeStruct((8, 128), jnp.float32),
        ],
        scratch_shapes=[pltpu.VMEM((2 * N_EXPERTS, tb), jnp.float32)],
        compiler_params=pltpu.CompilerParams(
            dimension_semantics=("arbitrary",),
        ),
    )(hs, w)


def _route_body(p_hbm, idx_hbm, wgt_hbm, p_v, idx_v, wgt_v):
    wid = lax.axis_index("s") * NUM_CORES + lax.axis_index("c")
    pltpu.sync_copy(p_hbm.at[wid], p_v)

    @plsc.parallel_loop(0, TOKEN_BLOCK // LANES, unroll=8)
    def chunk(j):
        sl = pl.ds(j * LANES, LANES)
        pe = [p_v[e, sl] for e in range(N_EXPERTS)]

        best = pe[0]
        bi = jnp.zeros((LANES,), jnp.int32)
        for e in range(1, N_EXPERTS):
            upd = pe[e] > best
            best = jnp.where(upd, pe[e], best)
            bi = jnp.where(upd, e, bi)

        b2 = jnp.full((LANES,), -1.0, jnp.float32)
        bi2 = jnp.zeros((LANES,), jnp.int32)
        for e in range(N_EXPERTS):
            upd = (pe[e] > b2) & (bi != e)
            b2 = jnp.where(upd, pe[e], b2)
            bi2 = jnp.where(upd, e, bi2)

        inv = 1.0 / (best + b2 + 1e-20)
        sl = pl.ds(j * LANES, LANES)
        idx_v[0, sl] = bi
        idx_v[1, sl] = bi2
        wgt_v[0, sl] = best * inv
        wgt_v[1, sl] = b2 * inv

    pltpu.sync_copy(idx_v, idx_hbm.at[wid])
    pltpu.sync_copy(wgt_v, wgt_hbm.at[wid])


def _route_sc(scores):
    nw = scores.shape[0]
    tpw = scores.shape[2]
    mesh = plsc.VectorSubcoreMesh(core_axis_name="c", subcore_axis_name="s")
    run = functools.partial(
        pl.kernel,
        mesh=mesh,
        out_type=[
            jax.ShapeDtypeStruct((nw, TOP_K, tpw), jnp.int32),
            jax.ShapeDtypeStruct((nw, TOP_K, tpw), jnp.float32),
        ],
        scratch_types=[
            pltpu.VMEM((N_EXPERTS, tpw), jnp.float32),
            pltpu.VMEM((TOP_K, tpw), jnp.int32),
            pltpu.VMEM((TOP_K, tpw), jnp.float32),
        ],
    )(_route_body)
    return run(scores)


def kernel(hidden_states, kernel):
    bsz, seq_len, h = hidden_states.shape
    t = bsz * seq_len
    hs = hidden_states.reshape(t, h)

    scores, aux = _scores_kernel(hs, kernel)
    idx2d, wgt2d = _route_sc(scores)

    topk_idx = idx2d.transpose(0, 2, 1).reshape(t, TOP_K)
    topk_weight = wgt2d.transpose(0, 2, 1).reshape(t, TOP_K)
    aux_loss = aux[0, 0]
    return (topk_idx, topk_weight, aux_loss)
